# Initial kernel scaffold; baseline (speedup 1.0000x reference)
import jax
import jax.numpy as jnp
from jax.experimental import pallas as pl


def kernel(x, adj, params):
    raise NotImplementedError("write your pallas kernel here")


# speedup vs baseline: 1.0337x; 1.0337x over previous
import functools

import jax
import jax.numpy as jnp
from jax import lax
from jax.experimental import pallas as pl
from jax.experimental.pallas import tpu as pltpu

F32 = jnp.float32
PAD = 128


def _act(h, act):
    if act == 'relu':
        return jnp.maximum(h, 0.0)
    if act == 'tanh':
        return jnp.tanh(h)
    if act == 'sigmoid':
        return jax.nn.sigmoid(h)
    return h


def _pad_cols(w, n=PAD):
    return jnp.pad(w, ((0, 0), (0, n - w.shape[1])))


def _pad_rows(w, n=PAD):
    return jnp.pad(w, ((0, n - w.shape[0]), (0, 0)))


def _mlp_chain(h, weights, biases, acts, bm=512):
    m, k0 = h.shape
    n_out = weights[-1].shape[1]
    nl = len(weights)

    def kern(h_ref, *refs):
        out_ref = refs[-1]
        cur = h_ref[...]
        for li in range(nl):
            w = refs[2 * li][...]
            b = refs[2 * li + 1][...]
            cur = jnp.dot(cur, w, preferred_element_type=F32) + b
            cur = _act(cur, acts[li])
        out_ref[...] = cur

    in_specs = [pl.BlockSpec((bm, k0), lambda i: (i, 0))]
    operands = [h]
    for w, b in zip(weights, biases):
        in_specs.append(pl.BlockSpec(w.shape, lambda i: (0, 0)))
        in_specs.append(pl.BlockSpec((1, w.shape[1]), lambda i: (0, 0)))
        operands.append(w)
        operands.append(b.reshape(1, -1))
    return pl.pallas_call(
        kern,
        grid=(m // bm,),
        in_specs=in_specs,
        out_specs=pl.BlockSpec((bm, n_out), lambda i: (i, 0)),
        out_shape=jax.ShapeDtypeStruct((m, n_out), F32),
    )(*operands)


def _gnn_layer(adj, h, w, act, bm=256):
    m, k = adj.shape
    n = w.shape[1]

    def kern(adj_ref, h_ref, w_ref, out_ref, t_ref):
        @pl.when(pl.program_id(0) == 0)
        def _():
            t_ref[...] = jnp.dot(h_ref[...], w_ref[...],
                                 preferred_element_type=F32)
        out_ref[...] = _act(
            jnp.dot(adj_ref[...], t_ref[...], preferred_element_type=F32),
            act)

    return pl.pallas_call(
        kern,
        grid=(m // bm,),
        in_specs=[pl.BlockSpec((bm, k), lambda i: (i, 0)),
                  pl.BlockSpec(h.shape, lambda i: (0, 0)),
                  pl.BlockSpec(w.shape, lambda i: (0, 0))],
        out_specs=pl.BlockSpec((bm, n), lambda i: (i, 0)),
        out_shape=jax.ShapeDtypeStruct((m, n), F32),
        scratch_shapes=[pltpu.VMEM((k, n), F32)],
    )(adj, h, w)


def _fuse_agg(adj, a, z_ae, z_igae, bm=256):
    m, k = adj.shape
    n = a.shape[1]

    def kern(adj_ref, a_ref, zae_ref, zig_ref, out_ref, t_ref):
        @pl.when(pl.program_id(0) == 0)
        def _():
            av = a_ref[...]
            t_ref[...] = av * zae_ref[...] + (1.0 - av) * zig_ref[...]
        out_ref[...] = jnp.dot(adj_ref[...], t_ref[...],
                               preferred_element_type=F32)

    return pl.pallas_call(
        kern,
        grid=(m // bm,),
        in_specs=[pl.BlockSpec((bm, k), lambda i: (i, 0)),
                  pl.BlockSpec(a.shape, lambda i: (0, 0)),
                  pl.BlockSpec(z_ae.shape, lambda i: (0, 0)),
                  pl.BlockSpec(z_igae.shape, lambda i: (0, 0))],
        out_specs=pl.BlockSpec((bm, n), lambda i: (i, 0)),
        out_shape=jax.ShapeDtypeStruct((m, n), F32),
        scratch_shapes=[pltpu.VMEM((k, n), F32)],
    )(adj, a, z_ae, z_igae)


def _attn(z_l, z_l_t, gamma_v, bm=512):
    m, d = z_l.shape

    def kern(zb_ref, zt_ref, zf_ref, g_ref, out_ref):
        zb = zb_ref[...]
        s = jnp.dot(zb, zt_ref[...], preferred_element_type=F32)
        s = s - jnp.max(s, axis=1, keepdims=True)
        e = jnp.exp(s)
        p = e / jnp.sum(e, axis=1, keepdims=True)
        zg = jnp.dot(p, zf_ref[...], preferred_element_type=F32)
        out_ref[...] = g_ref[0, 0] * zg + zb

    return pl.pallas_call(
        kern,
        grid=(m // bm,),
        in_specs=[pl.BlockSpec((bm, d), lambda i: (i, 0)),
                  pl.BlockSpec(z_l_t.shape, lambda i: (0, 0)),
                  pl.BlockSpec(z_l.shape, lambda i: (0, 0)),
                  pl.BlockSpec((1, PAD), lambda i: (0, 0))],
        out_specs=pl.BlockSpec((bm, d), lambda i: (i, 0)),
        out_shape=jax.ShapeDtypeStruct((m, d), F32),
    )(z_l, z_l_t, z_l, gamma_v)


def _zinb(z, wh, bh, wpi, bpi, wd, bd, wm, bm_, bm=512):
    m = z.shape[0]
    n4 = wpi.shape[1]

    def kern(z_ref, wh_ref, bh_ref, wpi_ref, bpi_ref, wd_ref, bd_ref,
             wm_ref, bm_ref, pi_ref, disp_ref, mean_ref):
        h = jnp.maximum(
            jnp.dot(z_ref[...], wh_ref[...], preferred_element_type=F32)
            + bh_ref[...], 0.0)
        pi_ref[...] = jax.nn.sigmoid(
            jnp.dot(h, wpi_ref[...], preferred_element_type=F32)
            + bpi_ref[...])
        d = jax.nn.softplus(
            jnp.dot(h, wd_ref[...], preferred_element_type=F32)
            + bd_ref[...])
        disp_ref[...] = jnp.clip(d, 1e-4, 1e4)
        mm = jnp.dot(h, wm_ref[...], preferred_element_type=F32) + bm_ref[...]
        mean_ref[...] = jnp.clip(jnp.exp(jnp.clip(mm, -15.0, 15.0)),
                                 1e-5, 1e6)

    full = lambda arr: pl.BlockSpec(arr.shape, lambda i: (0, 0))
    hidden = wh.shape[1]
    return pl.pallas_call(
        kern,
        grid=(m // bm,),
        in_specs=[pl.BlockSpec((bm, z.shape[1]), lambda i: (i, 0)),
                  full(wh), pl.BlockSpec((1, hidden), lambda i: (0, 0)),
                  full(wpi), pl.BlockSpec((1, n4), lambda i: (0, 0)),
                  full(wd), pl.BlockSpec((1, n4), lambda i: (0, 0)),
                  full(wm), pl.BlockSpec((1, n4), lambda i: (0, 0))],
        out_specs=[pl.BlockSpec((bm, n4), lambda i: (i, 0))] * 3,
        out_shape=[jax.ShapeDtypeStruct((m, n4), F32)] * 3,
    )(z, wh, bh.reshape(1, -1), wpi, bpi.reshape(1, -1),
      wd, bd.reshape(1, -1), wm, bm_.reshape(1, -1))


def _adj_hat(zi, zi_t, zh, zh_t, bm=256):
    m = zi.shape[0]

    def kern(zib_ref, zit_ref, zhb_ref, zht_ref, out_ref):
        s1 = jnp.dot(zib_ref[...], zit_ref[...], preferred_element_type=F32)
        s2 = jnp.dot(zhb_ref[...], zht_ref[...], preferred_element_type=F32)
        out_ref[...] = jax.nn.sigmoid(s1) + jax.nn.sigmoid(s2)

    return pl.pallas_call(
        kern,
        grid=(m // bm,),
        in_specs=[pl.BlockSpec((bm, zi.shape[1]), lambda i: (i, 0)),
                  pl.BlockSpec(zi_t.shape, lambda i: (0, 0)),
                  pl.BlockSpec((bm, zh.shape[1]), lambda i: (i, 0)),
                  pl.BlockSpec(zh_t.shape, lambda i: (0, 0))],
        out_specs=pl.BlockSpec((bm, m), lambda i: (i, 0)),
        out_shape=jax.ShapeDtypeStruct((m, m), F32),
    )(zi, zi_t, zh, zh_t)


def kernel(x, adj, params):
    p = params

    z_ae_p = _mlp_chain(
        x,
        [p['ae_enc_w0'], p['ae_enc_w1'], p['ae_enc_w2'],
         _pad_cols(p['ae_enc_w3'])],
        [p['ae_enc_b0'], p['ae_enc_b1'], p['ae_enc_b2'],
         _pad_cols(p['ae_enc_b3'].reshape(1, -1)).reshape(-1)],
        ['relu', 'relu', 'relu', 'none'])

    g = _gnn_layer(adj, x, p['gae_enc_w0'], 'tanh')
    g = _gnn_layer(adj, g, p['gae_enc_w1'], 'tanh')
    g = _gnn_layer(adj, g, p['gae_enc_w2'], 'tanh')
    z_igae_p = _gnn_layer(adj, g, _pad_cols(p['gae_enc_w3']), 'none')

    a_p = _pad_cols(p['a'])
    z_l_p = _fuse_agg(adj, a_p, z_ae_p, z_igae_p)
    gamma_v = jnp.broadcast_to(p['gamma'].reshape(1, 1), (1, PAD))
    z_tilde_p = _attn(z_l_p, z_l_p.T, gamma_v)

    pi, disp, mean = _zinb(
        z_tilde_p, _pad_rows(p['zinb_h_w']), p['zinb_h_b'],
        p['zinb_pi_w'], p['zinb_pi_b'],
        p['zinb_disp_w'], p['zinb_disp_b'],
        p['zinb_mean_w'], p['zinb_mean_b'])

    x_hat = _mlp_chain(
        z_tilde_p,
        [_pad_rows(p['ae_dec_w0']), p['ae_dec_w1'], p['ae_dec_w2'],
         p['ae_dec_w3']],
        [p['ae_dec_b0'], p['ae_dec_b1'], p['ae_dec_b2'], p['ae_dec_b3']],
        ['relu', 'relu', 'relu', 'none'])

    g = _gnn_layer(adj, z_tilde_p, _pad_rows(p['gae_dec_w0']), 'tanh')
    g = _gnn_layer(adj, g, p['gae_dec_w1'], 'tanh')
    g = _gnn_layer(adj, g, p['gae_dec_w2'], 'tanh')
    z_hat = _gnn_layer(adj, g, p['gae_dec_w3'], 'none')

    adj_hat = _adj_hat(z_igae_p, z_igae_p.T, z_hat, z_hat.T)

    z_ae = z_ae_p[:, :20]
    z_igae = z_igae_p[:, :20]
    z_tilde = z_tilde_p[:, :20]
    return (x_hat, z_hat, adj_hat, z_ae, z_igae, z_tilde, pi, disp, mean)


# speedup vs baseline: 1.0666x; 1.0319x over previous
import functools

import jax
import jax.numpy as jnp
from jax import lax
from jax.experimental import pallas as pl
from jax.experimental.pallas import tpu as pltpu

F32 = jnp.float32
BF16 = jnp.bfloat16
PAD = 128


def _act(h, act):
    if act == 'relu':
        return jnp.maximum(h, 0.0)
    if act == 'tanh':
        return jnp.tanh(h)
    if act == 'sigmoid':
        return jax.nn.sigmoid(h)
    return h


def _pad_cols(w, n=PAD):
    return jnp.pad(w, ((0, 0), (0, n - w.shape[1])))


def _pad_rows(w, n=PAD):
    return jnp.pad(w, ((0, n - w.shape[0]), (0, 0)))


def _mlp_chain(h, weights, biases, acts, bm=512):
    m, k0 = h.shape
    n_out = weights[-1].shape[1]
    nl = len(weights)

    def kern(h_ref, *refs):
        out_ref = refs[-1]
        cur = h_ref[...]
        for li in range(nl):
            w = refs[2 * li][...]
            b = refs[2 * li + 1][...]
            cur = jnp.dot(cur, w, preferred_element_type=F32) + b
            cur = _act(cur, acts[li])
        out_ref[...] = cur

    in_specs = [pl.BlockSpec((bm, k0), lambda i: (i, 0))]
    operands = [h]
    for w, b in zip(weights, biases):
        in_specs.append(pl.BlockSpec(w.shape, lambda i: (0, 0)))
        in_specs.append(pl.BlockSpec((1, w.shape[1]), lambda i: (0, 0)))
        operands.append(w)
        operands.append(b.reshape(1, -1))
    return pl.pallas_call(
        kern,
        grid=(m // bm,),
        in_specs=in_specs,
        out_specs=pl.BlockSpec((bm, n_out), lambda i: (i, 0)),
        out_shape=jax.ShapeDtypeStruct((m, n_out), F32),
    )(*operands)


def _gnn_layer(adj, h, w, act, bm=256, out_dtype=F32):
    m, k = adj.shape
    n = w.shape[1]

    def kern(adj_ref, h_ref, w_ref, out_ref, t_ref):
        @pl.when(pl.program_id(0) == 0)
        def _():
            t_ref[...] = jnp.dot(h_ref[...], w_ref[...],
                                 preferred_element_type=F32).astype(BF16)
        out_ref[...] = _act(
            jnp.dot(adj_ref[...], t_ref[...], preferred_element_type=F32),
            act).astype(out_dtype)

    return pl.pallas_call(
        kern,
        grid=(m // bm,),
        in_specs=[pl.BlockSpec((bm, k), lambda i: (i, 0)),
                  pl.BlockSpec(h.shape, lambda i: (0, 0)),
                  pl.BlockSpec(w.shape, lambda i: (0, 0))],
        out_specs=pl.BlockSpec((bm, n), lambda i: (i, 0)),
        out_shape=jax.ShapeDtypeStruct((m, n), out_dtype),
        scratch_shapes=[pltpu.VMEM((k, n), BF16)],
    )(adj, h, w)


def _fuse_agg(adj, a, z_ae, z_igae, bm=256):
    m, k = adj.shape
    n = a.shape[1]

    def kern(adj_ref, a_ref, zae_ref, zig_ref, out_ref, t_ref):
        @pl.when(pl.program_id(0) == 0)
        def _():
            av = a_ref[...]
            t_ref[...] = (av * zae_ref[...]
                          + (1.0 - av) * zig_ref[...]).astype(BF16)
        out_ref[...] = jnp.dot(adj_ref[...], t_ref[...],
                               preferred_element_type=F32)

    return pl.pallas_call(
        kern,
        grid=(m // bm,),
        in_specs=[pl.BlockSpec((bm, k), lambda i: (i, 0)),
                  pl.BlockSpec(a.shape, lambda i: (0, 0)),
                  pl.BlockSpec(z_ae.shape, lambda i: (0, 0)),
                  pl.BlockSpec(z_igae.shape, lambda i: (0, 0))],
        out_specs=pl.BlockSpec((bm, n), lambda i: (i, 0)),
        out_shape=jax.ShapeDtypeStruct((m, n), F32),
        scratch_shapes=[pltpu.VMEM((k, n), BF16)],
    )(adj, a, z_ae, z_igae)


def _attn(z_l, z_l_t, gamma_v, bm=512):
    m, d = z_l.shape

    def kern(zb_ref, zt_ref, zf_ref, g_ref, out_ref):
        zb = zb_ref[...]
        s = jnp.dot(zb, zt_ref[...], preferred_element_type=F32)
        s = s - jnp.max(s, axis=1, keepdims=True)
        e = jnp.exp(s)
        p = e / jnp.sum(e, axis=1, keepdims=True)
        zg = jnp.dot(p, zf_ref[...], preferred_element_type=F32)
        out_ref[...] = g_ref[0, 0] * zg + zb

    return pl.pallas_call(
        kern,
        grid=(m // bm,),
        in_specs=[pl.BlockSpec((bm, d), lambda i: (i, 0)),
                  pl.BlockSpec(z_l_t.shape, lambda i: (0, 0)),
                  pl.BlockSpec(z_l.shape, lambda i: (0, 0)),
                  pl.BlockSpec((1, PAD), lambda i: (0, 0))],
        out_specs=pl.BlockSpec((bm, d), lambda i: (i, 0)),
        out_shape=jax.ShapeDtypeStruct((m, d), F32),
    )(z_l, z_l_t, z_l, gamma_v)


def _zinb(z, wh, bh, wpi, bpi, wd, bd, wm, bm_, bm=512):
    m = z.shape[0]
    n4 = wpi.shape[1]

    def kern(z_ref, wh_ref, bh_ref, wpi_ref, bpi_ref, wd_ref, bd_ref,
             wm_ref, bm_ref, pi_ref, disp_ref, mean_ref):
        h = jnp.maximum(
            jnp.dot(z_ref[...], wh_ref[...], preferred_element_type=F32)
            + bh_ref[...], 0.0)
        pi_ref[...] = jax.nn.sigmoid(
            jnp.dot(h, wpi_ref[...], preferred_element_type=F32)
            + bpi_ref[...])
        d = jax.nn.softplus(
            jnp.dot(h, wd_ref[...], preferred_element_type=F32)
            + bd_ref[...])
        disp_ref[...] = jnp.clip(d, 1e-4, 1e4)
        mm = jnp.dot(h, wm_ref[...], preferred_element_type=F32) + bm_ref[...]
        mean_ref[...] = jnp.clip(jnp.exp(jnp.clip(mm, -15.0, 15.0)),
                                 1e-5, 1e6)

    full = lambda arr: pl.BlockSpec(arr.shape, lambda i: (0, 0))
    hidden = wh.shape[1]
    return pl.pallas_call(
        kern,
        grid=(m // bm,),
        in_specs=[pl.BlockSpec((bm, z.shape[1]), lambda i: (i, 0)),
                  full(wh), pl.BlockSpec((1, hidden), lambda i: (0, 0)),
                  full(wpi), pl.BlockSpec((1, n4), lambda i: (0, 0)),
                  full(wd), pl.BlockSpec((1, n4), lambda i: (0, 0)),
                  full(wm), pl.BlockSpec((1, n4), lambda i: (0, 0))],
        out_specs=[pl.BlockSpec((bm, n4), lambda i: (i, 0))] * 3,
        out_shape=[jax.ShapeDtypeStruct((m, n4), F32)] * 3,
    )(z, wh, bh.reshape(1, -1), wpi, bpi.reshape(1, -1),
      wd, bd.reshape(1, -1), wm, bm_.reshape(1, -1))


def _adj_hat(zi, zi_t, zh, zh_t, bm=256):
    m = zi.shape[0]

    def kern(zib_ref, zit_ref, zhb_ref, zht_ref, out_ref):
        s1 = jnp.dot(zib_ref[...], zit_ref[...], preferred_element_type=F32)
        s2 = jnp.dot(zhb_ref[...], zht_ref[...], preferred_element_type=F32)
        out_ref[...] = jax.nn.sigmoid(s1) + jax.nn.sigmoid(s2)

    return pl.pallas_call(
        kern,
        grid=(m // bm,),
        in_specs=[pl.BlockSpec((bm, zi.shape[1]), lambda i: (i, 0)),
                  pl.BlockSpec(zi_t.shape, lambda i: (0, 0)),
                  pl.BlockSpec((bm, zh.shape[1]), lambda i: (i, 0)),
                  pl.BlockSpec(zh_t.shape, lambda i: (0, 0))],
        out_specs=pl.BlockSpec((bm, m), lambda i: (i, 0)),
        out_shape=jax.ShapeDtypeStruct((m, m), F32),
    )(zi, zi_t, zh, zh_t)


def kernel(x, adj, params):
    p = params

    z_ae_p = _mlp_chain(
        x,
        [p['ae_enc_w0'], p['ae_enc_w1'], p['ae_enc_w2'],
         _pad_cols(p['ae_enc_w3'])],
        [p['ae_enc_b0'], p['ae_enc_b1'], p['ae_enc_b2'],
         _pad_cols(p['ae_enc_b3'].reshape(1, -1)).reshape(-1)],
        ['relu', 'relu', 'relu', 'none'])

    adj_bf = adj.astype(BF16)
    g = _gnn_layer(adj_bf, x.astype(BF16), p['gae_enc_w0'].astype(BF16),
                   'tanh', out_dtype=BF16)
    g = _gnn_layer(adj_bf, g, p['gae_enc_w1'].astype(BF16), 'tanh',
                   out_dtype=BF16)
    g = _gnn_layer(adj_bf, g, p['gae_enc_w2'].astype(BF16), 'tanh',
                   out_dtype=BF16)
    z_igae_p = _gnn_layer(adj_bf, g,
                          _pad_cols(p['gae_enc_w3']).astype(BF16), 'none')

    a_p = _pad_cols(p['a'])
    z_l_p = _fuse_agg(adj_bf, a_p, z_ae_p, z_igae_p)
    gamma_v = jnp.broadcast_to(p['gamma'].reshape(1, 1), (1, PAD))
    z_tilde_p = _attn(z_l_p, z_l_p.T, gamma_v)

    pi, disp, mean = _zinb(
        z_tilde_p, _pad_rows(p['zinb_h_w']), p['zinb_h_b'],
        p['zinb_pi_w'], p['zinb_pi_b'],
        p['zinb_disp_w'], p['zinb_disp_b'],
        p['zinb_mean_w'], p['zinb_mean_b'])

    x_hat = _mlp_chain(
        z_tilde_p,
        [_pad_rows(p['ae_dec_w0']), p['ae_dec_w1'], p['ae_dec_w2'],
         p['ae_dec_w3']],
        [p['ae_dec_b0'], p['ae_dec_b1'], p['ae_dec_b2'], p['ae_dec_b3']],
        ['relu', 'relu', 'relu', 'none'])

    g = _gnn_layer(adj_bf, z_tilde_p.astype(BF16),
                   _pad_rows(p['gae_dec_w0']).astype(BF16), 'tanh',
                   out_dtype=BF16)
    g = _gnn_layer(adj_bf, g, p['gae_dec_w1'].astype(BF16), 'tanh',
                   out_dtype=BF16)
    g = _gnn_layer(adj_bf, g, p['gae_dec_w2'].astype(BF16), 'tanh',
                   out_dtype=BF16)
    z_hat = _gnn_layer(adj_bf, g, p['gae_dec_w3'].astype(BF16), 'none')

    zi_bf = z_igae_p.astype(BF16)
    zh_bf = z_hat.astype(BF16)
    adj_hat = _adj_hat(zi_bf, zi_bf.T, zh_bf, zh_bf.T)

    z_ae = z_ae_p[:, :20]
    z_igae = z_igae_p[:, :20]
    z_tilde = z_tilde_p[:, :20]
    return (x_hat, z_hat, adj_hat, z_ae, z_igae, z_tilde, pi, disp, mean)
